# initial kernel scaffold (unmeasured)
import jax
import jax.numpy as jnp
from jax import lax
from jax.experimental import pallas as pl
from jax.experimental.pallas import tpu as pltpu

N_DEV = 16


def kernel(x, w_mat, scale_x, scale_w):
    k_glob, k_loc = x.shape
    m_loc = k_glob // N_DEV
    n = w_mat.shape[1]

    def body(x_ref, w_ref, sx_ref, sw_ref, out_ref,
             x8_ref, w8_ref, comm_ref, send_sems, recv_sems):
        my_i = lax.axis_index("i")

        x8_ref[...] = x_ref[...].astype(jnp.float8_e5m2)

        comm_ref[:, pl.ds(my_i * k_loc, k_loc)] = x8_ref[pl.ds(my_i * m_loc, m_loc), :]

        barrier_sem = pltpu.get_barrier_semaphore()
        for j in range(N_DEV):
            @pl.when(j != my_i)
            def _():
                pl.semaphore_signal(
                    barrier_sem, inc=1,
                    device_id=(j,), device_id_type=pl.DeviceIdType.MESH,
                )
        pl.semaphore_wait(barrier_sem, N_DEV - 1)

        for j in range(N_DEV):
            @pl.when(j != my_i)
            def _():
                pltpu.make_async_remote_copy(
                    src_ref=x8_ref.at[pl.ds(j * m_loc, m_loc), :],
                    dst_ref=comm_ref.at[:, pl.ds(my_i * k_loc, k_loc)],
                    send_sem=send_sems.at[j],
                    recv_sem=recv_sems.at[my_i],
                    device_id=(j,),
                    device_id_type=pl.DeviceIdType.MESH,
                ).start()

        w8_ref[...] = w_ref[...].astype(jnp.float8_e5m2)

        for j in range(N_DEV):
            @pl.when(j != my_i)
            def _():
                pltpu.make_async_remote_copy(
                    src_ref=x8_ref.at[pl.ds(j * m_loc, m_loc), :],
                    dst_ref=comm_ref.at[:, pl.ds(j * k_loc, k_loc)],
                    send_sem=send_sems.at[j],
                    recv_sem=recv_sems.at[j],
                    device_id=(j,),
                    device_id_type=pl.DeviceIdType.MESH,
                ).wait_recv()

        acc = jax.lax.dot_general(
            comm_ref[...], w8_ref[...],
            dimension_numbers=(((1,), (0,)), ((), ())),
            preferred_element_type=jnp.float32,
        )
        out_ref[...] = acc * (sx_ref[0] * sw_ref[0])

        for j in range(N_DEV):
            @pl.when(j != my_i)
            def _():
                pltpu.make_async_remote_copy(
                    src_ref=x8_ref.at[pl.ds(j * m_loc, m_loc), :],
                    dst_ref=comm_ref.at[:, pl.ds(my_i * k_loc, k_loc)],
                    send_sem=send_sems.at[j],
                    recv_sem=recv_sems.at[my_i],
                    device_id=(j,),
                    device_id_type=pl.DeviceIdType.MESH,
                ).wait_send()

    return pl.pallas_call(
        body,
        out_shape=jax.ShapeDtypeStruct((m_loc, n), jnp.float32),
        in_specs=[
            pl.BlockSpec(memory_space=pltpu.VMEM),
            pl.BlockSpec(memory_space=pltpu.VMEM),
            pl.BlockSpec(memory_space=pltpu.SMEM),
            pl.BlockSpec(memory_space=pltpu.SMEM),
        ],
        out_specs=pl.BlockSpec(memory_space=pltpu.VMEM),
        scratch_shapes=[
            pltpu.VMEM((k_glob, k_loc), jnp.float8_e5m2),
            pltpu.VMEM((k_glob, n), jnp.float8_e5m2),
            pltpu.VMEM((m_loc, k_glob), jnp.float8_e5m2),
            pltpu.SemaphoreType.DMA((N_DEV,)),
            pltpu.SemaphoreType.DMA((N_DEV,)),
        ],
        compiler_params=pltpu.CompilerParams(collective_id=0),
    )(x, w_mat, scale_x, scale_w)


# baseline (device time: 34749 ns/iter reference)
import jax
import jax.numpy as jnp
from jax import lax
from jax.experimental import pallas as pl
from jax.experimental.pallas import tpu as pltpu

N_DEV = 16


def kernel(x, w_mat, scale_x, scale_w):
    k_glob, k_loc = x.shape
    m_loc = k_glob // N_DEV
    n = w_mat.shape[1]

    def body(x_ref, w_ref, sx_ref, sw_ref, out_ref,
             x8_ref, w8_ref, comm_ref, send_sems, recv_sems):
        my_i = lax.axis_index("i")

        x8_ref[...] = x_ref[...].astype(jnp.float8_e5m2)

        comm_ref[:, pl.ds(my_i * k_loc, k_loc)] = x8_ref[pl.ds(my_i * m_loc, m_loc), :]

        barrier_sem = pltpu.get_barrier_semaphore()
        for j in range(N_DEV):
            @pl.when(j != my_i)
            def _():
                pl.semaphore_signal(
                    barrier_sem, inc=1,
                    device_id=(j,), device_id_type=pl.DeviceIdType.MESH,
                )
        pl.semaphore_wait(barrier_sem, N_DEV - 1)

        for j in range(N_DEV):
            @pl.when(j != my_i)
            def _():
                pltpu.make_async_remote_copy(
                    src_ref=x8_ref.at[pl.ds(j * m_loc, m_loc), :],
                    dst_ref=comm_ref.at[:, pl.ds(my_i * k_loc, k_loc)],
                    send_sem=send_sems.at[j],
                    recv_sem=recv_sems.at[my_i],
                    device_id=(j,),
                    device_id_type=pl.DeviceIdType.MESH,
                ).start()

        w8_ref[...] = w_ref[...].astype(jnp.float8_e5m2)

        for j in range(N_DEV):
            @pl.when(j != my_i)
            def _():
                pltpu.make_async_remote_copy(
                    src_ref=x8_ref.at[pl.ds(j * m_loc, m_loc), :],
                    dst_ref=comm_ref.at[:, pl.ds(j * k_loc, k_loc)],
                    send_sem=send_sems.at[j],
                    recv_sem=recv_sems.at[j],
                    device_id=(j,),
                    device_id_type=pl.DeviceIdType.MESH,
                ).wait_recv()

        acc = jax.lax.dot_general(
            comm_ref[...], w8_ref[...],
            dimension_numbers=(((1,), (0,)), ((), ())),
            preferred_element_type=jnp.float32,
        )
        out_ref[...] = acc * (sx_ref[0] * sw_ref[0])

        for j in range(N_DEV):
            @pl.when(j != my_i)
            def _():
                pltpu.make_async_remote_copy(
                    src_ref=x8_ref.at[pl.ds(j * m_loc, m_loc), :],
                    dst_ref=comm_ref.at[:, pl.ds(my_i * k_loc, k_loc)],
                    send_sem=send_sems.at[j],
                    recv_sem=recv_sems.at[my_i],
                    device_id=(j,),
                    device_id_type=pl.DeviceIdType.MESH,
                ).wait_send()

    return pl.pallas_call(
        body,
        out_shape=jax.ShapeDtypeStruct((m_loc, n), jnp.float32),
        in_specs=[
            pl.BlockSpec(memory_space=pltpu.VMEM),
            pl.BlockSpec(memory_space=pltpu.VMEM),
            pl.BlockSpec(memory_space=pltpu.SMEM),
            pl.BlockSpec(memory_space=pltpu.SMEM),
        ],
        out_specs=pl.BlockSpec(memory_space=pltpu.VMEM),
        scratch_shapes=[
            pltpu.VMEM((k_glob, k_loc), jnp.float8_e5m2),
            pltpu.VMEM((k_glob, n), jnp.float8_e5m2),
            pltpu.VMEM((m_loc, k_glob), jnp.float8_e5m2),
            pltpu.SemaphoreType.DMA((N_DEV,)),
            pltpu.SemaphoreType.DMA((N_DEV,)),
        ],
        compiler_params=pltpu.CompilerParams(
            collective_id=0, vmem_limit_bytes=100 * 1024 * 1024,
        ),
    )(x, w_mat, scale_x, scale_w)


# device time: 24940 ns/iter; 1.3933x vs baseline; 1.3933x over previous
import jax
import jax.numpy as jnp
from jax import lax
from jax.experimental import pallas as pl
from jax.experimental.pallas import tpu as pltpu

N_DEV = 16
N_CHUNKS = 4
PER_CHUNK = N_DEV // N_CHUNKS


def kernel(x, w_mat, scale_x, scale_w):
    k_glob, k_loc = x.shape
    m_loc = k_glob // N_DEV
    n = w_mat.shape[1]
    k_chunk = k_glob // N_CHUNKS

    def body(x_ref, w_ref, sx_ref, sw_ref, out_ref,
             x8_ref, comm_ref, wf_ref, w8_ref, send_sems, recv_sems, w_sems):
        my_i = lax.axis_index("i")

        def w_dma(c):
            return pltpu.make_async_copy(
                w_ref.at[pl.ds(c * k_chunk, k_chunk), :],
                wf_ref.at[c % 2],
                w_sems.at[c % 2],
            )

        w_dma(0).start()
        w_dma(1).start()

        x8_ref[...] = x_ref[...].astype(jnp.float8_e5m2)

        comm_ref[:, pl.ds(my_i * k_loc, k_loc)] = x8_ref[pl.ds(my_i * m_loc, m_loc), :]

        barrier_sem = pltpu.get_barrier_semaphore()
        for j in range(N_DEV):
            @pl.when(j != my_i)
            def _():
                pl.semaphore_signal(
                    barrier_sem, inc=1,
                    device_id=(j,), device_id_type=pl.DeviceIdType.MESH,
                )
        pl.semaphore_wait(barrier_sem, N_DEV - 1)

        for j in range(N_DEV):
            @pl.when(j != my_i)
            def _():
                pltpu.make_async_remote_copy(
                    src_ref=x8_ref.at[pl.ds(j * m_loc, m_loc), :],
                    dst_ref=comm_ref.at[:, pl.ds(my_i * k_loc, k_loc)],
                    send_sem=send_sems.at[j],
                    recv_sem=recv_sems.at[my_i],
                    device_id=(j,),
                    device_id_type=pl.DeviceIdType.MESH,
                ).start()

        s = sx_ref[0] * sw_ref[0]
        for c in range(N_CHUNKS):
            b = c % 2
            w_dma(c).wait()
            w8_ref[b] = wf_ref[b].astype(jnp.float8_e5m2)
            if c + 2 < N_CHUNKS:
                w_dma(c + 2).start()
            for j in range(c * PER_CHUNK, (c + 1) * PER_CHUNK):
                @pl.when(j != my_i)
                def _():
                    pltpu.make_async_remote_copy(
                        src_ref=x8_ref.at[pl.ds(j * m_loc, m_loc), :],
                        dst_ref=comm_ref.at[:, pl.ds(j * k_loc, k_loc)],
                        send_sem=send_sems.at[j],
                        recv_sem=recv_sems.at[j],
                        device_id=(j,),
                        device_id_type=pl.DeviceIdType.MESH,
                    ).wait_recv()
            partial = jax.lax.dot_general(
                comm_ref[:, pl.ds(c * k_chunk, k_chunk)], w8_ref[b],
                dimension_numbers=(((1,), (0,)), ((), ())),
                preferred_element_type=jnp.float32,
            )
            if c == 0:
                out_ref[...] = partial
            elif c < N_CHUNKS - 1:
                out_ref[...] += partial
            else:
                out_ref[...] = (out_ref[...] + partial) * s

        for j in range(N_DEV):
            @pl.when(j != my_i)
            def _():
                pltpu.make_async_remote_copy(
                    src_ref=x8_ref.at[pl.ds(j * m_loc, m_loc), :],
                    dst_ref=comm_ref.at[:, pl.ds(my_i * k_loc, k_loc)],
                    send_sem=send_sems.at[j],
                    recv_sem=recv_sems.at[my_i],
                    device_id=(j,),
                    device_id_type=pl.DeviceIdType.MESH,
                ).wait_send()

    return pl.pallas_call(
        body,
        out_shape=jax.ShapeDtypeStruct((m_loc, n), jnp.float32),
        in_specs=[
            pl.BlockSpec(memory_space=pltpu.VMEM),
            pl.BlockSpec(memory_space=pltpu.MemorySpace.HBM),
            pl.BlockSpec(memory_space=pltpu.SMEM),
            pl.BlockSpec(memory_space=pltpu.SMEM),
        ],
        out_specs=pl.BlockSpec(memory_space=pltpu.VMEM),
        scratch_shapes=[
            pltpu.VMEM((k_glob, k_loc), jnp.float8_e5m2),
            pltpu.VMEM((m_loc, k_glob), jnp.float8_e5m2),
            pltpu.VMEM((2, k_chunk, n), jnp.float32),
            pltpu.VMEM((2, k_chunk, n), jnp.float8_e5m2),
            pltpu.SemaphoreType.DMA((N_DEV,)),
            pltpu.SemaphoreType.DMA((N_DEV,)),
            pltpu.SemaphoreType.DMA((2,)),
        ],
        compiler_params=pltpu.CompilerParams(
            collective_id=0, vmem_limit_bytes=100 * 1024 * 1024,
        ),
    )(x, w_mat, scale_x, scale_w)


# device time: 24918 ns/iter; 1.3945x vs baseline; 1.0009x over previous
import jax
import jax.numpy as jnp
from jax import lax
from jax.experimental import pallas as pl
from jax.experimental.pallas import tpu as pltpu

N_DEV = 16
N_CHUNKS = 4
PER_CHUNK = N_DEV // N_CHUNKS


def kernel(x, w_mat, scale_x, scale_w):
    k_glob, k_loc = x.shape
    m_loc = k_glob // N_DEV
    n = w_mat.shape[1]
    k_chunk = k_glob // N_CHUNKS

    def body(x_ref, w_ref, sx_ref, sw_ref, out_ref,
             x8_ref, comm_ref, wf_ref, w8_ref, send_sems, recv_sems, w_sems):
        my_i = lax.axis_index("i")
        my_chunk = my_i // PER_CHUNK

        def chunk_at(t):
            return lax.rem(my_chunk + t, N_CHUNKS)

        def w_dma(t):
            cc = chunk_at(t)
            return pltpu.make_async_copy(
                w_ref.at[pl.ds(cc * k_chunk, k_chunk), :],
                wf_ref.at[t % 2],
                w_sems.at[t % 2],
            )

        w_dma(0).start()
        w_dma(1).start()

        x8_ref[...] = x_ref[...].astype(jnp.float8_e5m2)

        comm_ref[:, pl.ds(my_i * k_loc, k_loc)] = x8_ref[pl.ds(my_i * m_loc, m_loc), :]

        barrier_sem = pltpu.get_barrier_semaphore()
        for j in range(N_DEV):
            @pl.when(j != my_i)
            def _():
                pl.semaphore_signal(
                    barrier_sem, inc=1,
                    device_id=(j,), device_id_type=pl.DeviceIdType.MESH,
                )
        pl.semaphore_wait(barrier_sem, N_DEV - 1)

        for d in range(1, N_DEV):
            j = lax.rem(my_i + d, N_DEV)
            pltpu.make_async_remote_copy(
                src_ref=x8_ref.at[pl.ds(j * m_loc, m_loc), :],
                dst_ref=comm_ref.at[:, pl.ds(my_i * k_loc, k_loc)],
                send_sem=send_sems.at[d],
                recv_sem=recv_sems.at[my_i],
                device_id=(j,),
                device_id_type=pl.DeviceIdType.MESH,
            ).start()

        s = sx_ref[0] * sw_ref[0]
        for t in range(N_CHUNKS):
            b = t % 2
            cc = chunk_at(t)
            w_dma(t).wait()
            w8_ref[b] = wf_ref[b].astype(jnp.float8_e5m2)
            if t + 2 < N_CHUNKS:
                w_dma(t + 2).start()
            for u in range(PER_CHUNK):
                j = cc * PER_CHUNK + u
                @pl.when(j != my_i)
                def _():
                    pltpu.make_async_remote_copy(
                        src_ref=x8_ref.at[pl.ds(j * m_loc, m_loc), :],
                        dst_ref=comm_ref.at[:, pl.ds(j * k_loc, k_loc)],
                        send_sem=send_sems.at[0],
                        recv_sem=recv_sems.at[j],
                        device_id=(j,),
                        device_id_type=pl.DeviceIdType.MESH,
                    ).wait_recv()
            partial = jax.lax.dot_general(
                comm_ref[:, pl.ds(cc * k_chunk, k_chunk)], w8_ref[b],
                dimension_numbers=(((1,), (0,)), ((), ())),
                preferred_element_type=jnp.float32,
            )
            if t == 0:
                out_ref[...] = partial
            elif t < N_CHUNKS - 1:
                out_ref[...] += partial
            else:
                out_ref[...] = (out_ref[...] + partial) * s

        for d in range(1, N_DEV):
            j = lax.rem(my_i + d, N_DEV)
            pltpu.make_async_remote_copy(
                src_ref=x8_ref.at[pl.ds(j * m_loc, m_loc), :],
                dst_ref=comm_ref.at[:, pl.ds(my_i * k_loc, k_loc)],
                send_sem=send_sems.at[d],
                recv_sem=recv_sems.at[my_i],
                device_id=(j,),
                device_id_type=pl.DeviceIdType.MESH,
            ).wait_send()

    return pl.pallas_call(
        body,
        out_shape=jax.ShapeDtypeStruct((m_loc, n), jnp.float32),
        in_specs=[
            pl.BlockSpec(memory_space=pltpu.VMEM),
            pl.BlockSpec(memory_space=pltpu.MemorySpace.HBM),
            pl.BlockSpec(memory_space=pltpu.SMEM),
            pl.BlockSpec(memory_space=pltpu.SMEM),
        ],
        out_specs=pl.BlockSpec(memory_space=pltpu.VMEM),
        scratch_shapes=[
            pltpu.VMEM((k_glob, k_loc), jnp.float8_e5m2),
            pltpu.VMEM((m_loc, k_glob), jnp.float8_e5m2),
            pltpu.VMEM((2, k_chunk, n), jnp.float32),
            pltpu.VMEM((2, k_chunk, n), jnp.float8_e5m2),
            pltpu.SemaphoreType.DMA((N_DEV,)),
            pltpu.SemaphoreType.DMA((N_DEV,)),
            pltpu.SemaphoreType.DMA((2,)),
        ],
        compiler_params=pltpu.CompilerParams(
            collective_id=0, vmem_limit_bytes=100 * 1024 * 1024,
        ),
    )(x, w_mat, scale_x, scale_w)


# device time: 22491 ns/iter; 1.5450x vs baseline; 1.1079x over previous
import jax
import jax.numpy as jnp
from jax import lax
from jax.experimental import pallas as pl
from jax.experimental.pallas import tpu as pltpu

N_DEV = 16
N_CHUNKS = 4
PER_CHUNK = N_DEV // N_CHUNKS


def kernel(x, w_mat, scale_x, scale_w):
    k_glob, k_loc = x.shape
    m_loc = k_glob // N_DEV
    n = w_mat.shape[1]
    k_chunk = k_glob // N_CHUNKS

    def body(x_ref, w_ref, sx_ref, sw_ref, out_ref,
             xf_ref, x8_ref, comm_ref, wf_ref, w8_ref,
             send_sems, recv_sems, w_sems, x_sems):
        my_i = lax.axis_index("i")
        my_chunk = my_i // PER_CHUNK

        def chunk_at(t):
            return lax.rem(my_chunk + t, N_CHUNKS)

        def w_dma(t):
            cc = chunk_at(t)
            return pltpu.make_async_copy(
                w_ref.at[pl.ds(cc * k_chunk, k_chunk), :],
                wf_ref.at[t % 2],
                w_sems.at[t % 2],
            )

        m_plane = PER_CHUNK * m_loc

        def x_dma(s):
            q = chunk_at(s)
            return pltpu.make_async_copy(
                x_ref.at[pl.ds(q * m_plane, m_plane), :],
                xf_ref.at[pl.ds(q * m_plane, m_plane), :],
                x_sems.at[s],
            )

        for s in range(N_CHUNKS):
            x_dma(s).start()
        w_dma(0).start()
        w_dma(1).start()

        barrier_sem = pltpu.get_barrier_semaphore()
        for j in range(N_DEV):
            @pl.when(j != my_i)
            def _():
                pl.semaphore_signal(
                    barrier_sem, inc=1,
                    device_id=(j,), device_id_type=pl.DeviceIdType.MESH,
                )
        pl.semaphore_wait(barrier_sem, N_DEV - 1)

        for s in range(N_CHUNKS):
            q = chunk_at(s)
            x_dma(s).wait()
            rows = pl.ds(q * m_plane, m_plane)
            x8_ref[rows, :] = xf_ref[rows, :].astype(jnp.float8_e5m2)
            for u in range(PER_CHUNK):
                j = q * PER_CHUNK + u
                @pl.when(j == my_i)
                def _():
                    comm_ref[:, pl.ds(my_i * k_loc, k_loc)] = (
                        x8_ref[pl.ds(my_i * m_loc, m_loc), :])
                @pl.when(j != my_i)
                def _():
                    pltpu.make_async_remote_copy(
                        src_ref=x8_ref.at[pl.ds(j * m_loc, m_loc), :],
                        dst_ref=comm_ref.at[:, pl.ds(my_i * k_loc, k_loc)],
                        send_sem=send_sems.at[j],
                        recv_sem=recv_sems.at[my_i],
                        device_id=(j,),
                        device_id_type=pl.DeviceIdType.MESH,
                    ).start()

        s = sx_ref[0] * sw_ref[0]
        for t in range(N_CHUNKS):
            b = t % 2
            cc = chunk_at(t)
            w_dma(t).wait()
            w8_ref[b] = wf_ref[b].astype(jnp.float8_e5m2)
            if t + 2 < N_CHUNKS:
                w_dma(t + 2).start()
            for u in range(PER_CHUNK):
                j = cc * PER_CHUNK + u
                @pl.when(j != my_i)
                def _():
                    pltpu.make_async_remote_copy(
                        src_ref=x8_ref.at[pl.ds(j * m_loc, m_loc), :],
                        dst_ref=comm_ref.at[:, pl.ds(j * k_loc, k_loc)],
                        send_sem=send_sems.at[0],
                        recv_sem=recv_sems.at[j],
                        device_id=(j,),
                        device_id_type=pl.DeviceIdType.MESH,
                    ).wait_recv()
            partial = jax.lax.dot_general(
                comm_ref[:, pl.ds(cc * k_chunk, k_chunk)], w8_ref[b],
                dimension_numbers=(((1,), (0,)), ((), ())),
                preferred_element_type=jnp.float32,
            )
            if t == 0:
                out_ref[...] = partial
            elif t < N_CHUNKS - 1:
                out_ref[...] += partial
            else:
                out_ref[...] = (out_ref[...] + partial) * s

        for j in range(N_DEV):
            @pl.when(j != my_i)
            def _():
                pltpu.make_async_remote_copy(
                    src_ref=x8_ref.at[pl.ds(j * m_loc, m_loc), :],
                    dst_ref=comm_ref.at[:, pl.ds(my_i * k_loc, k_loc)],
                    send_sem=send_sems.at[j],
                    recv_sem=recv_sems.at[my_i],
                    device_id=(j,),
                    device_id_type=pl.DeviceIdType.MESH,
                ).wait_send()

    return pl.pallas_call(
        body,
        out_shape=jax.ShapeDtypeStruct((m_loc, n), jnp.float32),
        in_specs=[
            pl.BlockSpec(memory_space=pltpu.MemorySpace.HBM),
            pl.BlockSpec(memory_space=pltpu.MemorySpace.HBM),
            pl.BlockSpec(memory_space=pltpu.SMEM),
            pl.BlockSpec(memory_space=pltpu.SMEM),
        ],
        out_specs=pl.BlockSpec(memory_space=pltpu.VMEM),
        scratch_shapes=[
            pltpu.VMEM((k_glob, k_loc), jnp.float32),
            pltpu.VMEM((k_glob, k_loc), jnp.float8_e5m2),
            pltpu.VMEM((m_loc, k_glob), jnp.float8_e5m2),
            pltpu.VMEM((2, k_chunk, n), jnp.float32),
            pltpu.VMEM((2, k_chunk, n), jnp.float8_e5m2),
            pltpu.SemaphoreType.DMA((N_DEV,)),
            pltpu.SemaphoreType.DMA((N_DEV,)),
            pltpu.SemaphoreType.DMA((2,)),
            pltpu.SemaphoreType.DMA((N_CHUNKS,)),
        ],
        compiler_params=pltpu.CompilerParams(
            collective_id=0, vmem_limit_bytes=100 * 1024 * 1024,
        ),
    )(x, w_mat, scale_x, scale_w)


# device time: 22368 ns/iter; 1.5535x vs baseline; 1.0055x over previous
import jax
import jax.numpy as jnp
from jax import lax
from jax.experimental import pallas as pl
from jax.experimental.pallas import tpu as pltpu

N_DEV = 16
N_CHUNKS = 4
PER_CHUNK = N_DEV // N_CHUNKS


def kernel(x, w_mat, scale_x, scale_w):
    k_glob, k_loc = x.shape
    m_loc = k_glob // N_DEV
    n = w_mat.shape[1]
    k_chunk = k_glob // N_CHUNKS

    def body(x_ref, w_ref, sx_ref, sw_ref, out_ref,
             xf_ref, x8_ref, comm_ref, wf_ref, w8_ref,
             send_sems, recv_sems, w_sems, x_sems):
        my_i = lax.axis_index("i")
        my_chunk = my_i // PER_CHUNK

        def chunk_at(t):
            return lax.rem(my_chunk + t, N_CHUNKS)

        def w_dma(t):
            cc = chunk_at(t)
            return pltpu.make_async_copy(
                w_ref.at[pl.ds(cc * k_chunk, k_chunk), :],
                wf_ref.at[t % 2],
                w_sems.at[t % 2],
            )

        m_plane = PER_CHUNK * m_loc

        def x_dma(s):
            q = chunk_at(s)
            return pltpu.make_async_copy(
                x_ref.at[pl.ds(q * m_plane, m_plane), :],
                xf_ref.at[pl.ds(q * m_plane, m_plane), :],
                x_sems.at[s],
            )

        for s in range(N_CHUNKS):
            x_dma(s).start()
        w_dma(0).start()
        w_dma(1).start()

        barrier_sem = pltpu.get_barrier_semaphore()
        for j in range(N_DEV):
            @pl.when(j != my_i)
            def _():
                pl.semaphore_signal(
                    barrier_sem, inc=1,
                    device_id=(j,), device_id_type=pl.DeviceIdType.MESH,
                )

        for s in range(N_CHUNKS):
            q = chunk_at(s)
            x_dma(s).wait()
            rows = pl.ds(q * m_plane, m_plane)
            x8_ref[rows, :] = xf_ref[rows, :].astype(jnp.float8_e5m2)
            if s == 0:
                pl.semaphore_wait(barrier_sem, N_DEV - 1)
            for u in range(PER_CHUNK):
                j = q * PER_CHUNK + u
                @pl.when(j == my_i)
                def _():
                    comm_ref[:, pl.ds(my_i * k_loc, k_loc)] = (
                        x8_ref[pl.ds(my_i * m_loc, m_loc), :])
                @pl.when(j != my_i)
                def _():
                    pltpu.make_async_remote_copy(
                        src_ref=x8_ref.at[pl.ds(j * m_loc, m_loc), :],
                        dst_ref=comm_ref.at[:, pl.ds(my_i * k_loc, k_loc)],
                        send_sem=send_sems.at[j],
                        recv_sem=recv_sems.at[my_i],
                        device_id=(j,),
                        device_id_type=pl.DeviceIdType.MESH,
                    ).start()

        s = sx_ref[0] * sw_ref[0]
        for t in range(N_CHUNKS):
            b = t % 2
            cc = chunk_at(t)
            w_dma(t).wait()
            w8_ref[b] = wf_ref[b].astype(jnp.float8_e5m2)
            if t + 2 < N_CHUNKS:
                w_dma(t + 2).start()
            for u in range(PER_CHUNK):
                j = cc * PER_CHUNK + u
                @pl.when(j != my_i)
                def _():
                    pltpu.make_async_remote_copy(
                        src_ref=x8_ref.at[pl.ds(j * m_loc, m_loc), :],
                        dst_ref=comm_ref.at[:, pl.ds(j * k_loc, k_loc)],
                        send_sem=send_sems.at[0],
                        recv_sem=recv_sems.at[j],
                        device_id=(j,),
                        device_id_type=pl.DeviceIdType.MESH,
                    ).wait_recv()
            partial = jax.lax.dot_general(
                comm_ref[:, pl.ds(cc * k_chunk, k_chunk)], w8_ref[b],
                dimension_numbers=(((1,), (0,)), ((), ())),
                preferred_element_type=jnp.float32,
            )
            if t == 0:
                out_ref[...] = partial
            elif t < N_CHUNKS - 1:
                out_ref[...] += partial
            else:
                out_ref[...] = (out_ref[...] + partial) * s

        for j in range(N_DEV):
            @pl.when(j != my_i)
            def _():
                pltpu.make_async_remote_copy(
                    src_ref=x8_ref.at[pl.ds(j * m_loc, m_loc), :],
                    dst_ref=comm_ref.at[:, pl.ds(my_i * k_loc, k_loc)],
                    send_sem=send_sems.at[j],
                    recv_sem=recv_sems.at[my_i],
                    device_id=(j,),
                    device_id_type=pl.DeviceIdType.MESH,
                ).wait_send()

    return pl.pallas_call(
        body,
        out_shape=jax.ShapeDtypeStruct((m_loc, n), jnp.float32),
        in_specs=[
            pl.BlockSpec(memory_space=pltpu.MemorySpace.HBM),
            pl.BlockSpec(memory_space=pltpu.MemorySpace.HBM),
            pl.BlockSpec(memory_space=pltpu.SMEM),
            pl.BlockSpec(memory_space=pltpu.SMEM),
        ],
        out_specs=pl.BlockSpec(memory_space=pltpu.VMEM),
        scratch_shapes=[
            pltpu.VMEM((k_glob, k_loc), jnp.float32),
            pltpu.VMEM((k_glob, k_loc), jnp.float8_e5m2),
            pltpu.VMEM((m_loc, k_glob), jnp.float8_e5m2),
            pltpu.VMEM((2, k_chunk, n), jnp.float32),
            pltpu.VMEM((2, k_chunk, n), jnp.float8_e5m2),
            pltpu.SemaphoreType.DMA((N_DEV,)),
            pltpu.SemaphoreType.DMA((N_DEV,)),
            pltpu.SemaphoreType.DMA((2,)),
            pltpu.SemaphoreType.DMA((N_CHUNKS,)),
        ],
        compiler_params=pltpu.CompilerParams(
            collective_id=0, vmem_limit_bytes=100 * 1024 * 1024,
        ),
    )(x, w_mat, scale_x, scale_w)
